# Initial kernel scaffold; baseline (speedup 1.0000x reference)
import jax
import jax.numpy as jnp
from jax.experimental import pallas as pl


def kernel(input_ids, token_type_ids, word_table, pos_table, type_table, gamma, beta):
    raise NotImplementedError("write your pallas kernel here")


# speedup vs baseline: 2.9757x; 2.9757x over previous
import functools

import jax
import jax.numpy as jnp
from jax import lax
from jax.experimental import pallas as pl
from jax.experimental.pallas import tpu as pltpu
from jax.experimental.pallas import tpu_sc as plsc

_F32 = jnp.float32
_I32 = jnp.int32
_EPS = 1e-12
_IDXW = 100


@functools.lru_cache(maxsize=None)
def _make_sc_kernel(TOK, S, H, V):
    info = plsc.get_sparse_core_info()
    NC, NS, L = info.num_cores, info.num_subcores, info.num_lanes
    NW = NC * NS
    assert TOK % (NW * S) == 0
    per_w = TOK // NW
    chunks = per_w // S
    KH = H // L
    assert S % _IDXW == 0
    G = S // _IDXW
    rows_per_w = per_w // _IDXW

    mesh = plsc.VectorSubcoreMesh(core_axis_name="c", subcore_axis_name="s")

    @functools.partial(
        pl.kernel,
        mesh=mesh,
        out_type=jax.ShapeDtypeStruct((TOK, H), _F32),
        scratch_types=[
            pltpu.VMEM((G, _IDXW), _I32),
            pltpu.VMEM((S + 16,), _I32),
            pltpu.VMEM((S, H), _F32),
            pltpu.VMEM((S, H), _F32),
            pltpu.VMEM((2, H), _F32),
            pltpu.VMEM((H,), _F32),
            pltpu.VMEM((H,), _F32),
            pltpu.VMEM((H,), _F32),
            pltpu.SemaphoreType.DMA,
        ],
    )
    def sc_kernel(ids_hbm, tt_hbm, word_hbm, pos_hbm, type_hbm, gamma_hbm,
                  beta_hbm, out_hbm, idx_v, tt_v, posx_v, rows_v, type_v,
                  gamma_v, beta_v, dtv_v, sem):
        wid = lax.axis_index("s") * NC + lax.axis_index("c")

        pltpu.sync_copy(pos_hbm.at[pl.ds(0, S)], posx_v)
        pltpu.sync_copy(type_hbm, type_v)
        pltpu.sync_copy(gamma_hbm, gamma_v)
        pltpu.sync_copy(beta_hbm, beta_v)
        for k in range(KH):
            sl = pl.ds(k * L, L)
            dtv_v[sl] = type_v[1, sl] - type_v[0, sl]

        def fuse_body(j, carry):
            for k in range(KH):
                sl = pl.ds(k * L, L)
                posx_v[j, sl] = posx_v[j, sl] + type_v[0, sl]
            return carry

        lax.fori_loop(0, S, fuse_body, 0)

        def chunk_body(c, carry):
            base = wid * per_w + c * S
            pltpu.sync_copy(tt_hbm.at[pl.ds(base, S)], tt_v.at[pl.ds(0, S)])
            rbase = wid * rows_per_w + c * G
            pltpu.sync_copy(ids_hbm.at[pl.ds(rbase, G)], idx_v)
            copies = [
                pltpu.async_copy(word_hbm.at[idx_v.at[g]],
                                 rows_v.at[pl.ds(g * _IDXW, _IDXW)], sem)
                for g in range(G)
            ]
            for cp in copies:
                cp.wait()

            def tok_body(j, tcarry):
                ttf = jnp.full((L,), tt_v[pl.ds(j, L)][0].astype(_F32))
                s = jnp.zeros((L,), _F32)
                q = jnp.zeros((L,), _F32)
                r = []
                for k in range(KH):
                    sl = pl.ds(k * L, L)
                    a = rows_v[j, sl] + posx_v[j, sl] + ttf * dtv_v[sl]
                    r.append(a)
                    s = s + a
                    q = q + a * a
                lanes = jnp.arange(L, dtype=_I32)
                for d in (8, 4, 2, 1):
                    perm = lanes ^ d
                    s = s + jnp.take_along_axis(
                        s, perm, axis=0, mode="promise_in_bounds")
                    q = q + jnp.take_along_axis(
                        q, perm, axis=0, mode="promise_in_bounds")
                meanv = s * (1.0 / H)
                qv = q
                varv = qv * (1.0 / H) - meanv * meanv
                x = varv + _EPS
                xi = lax.bitcast_convert_type(x, _I32)
                yi = jnp.int32(0x5F3759DF) - lax.shift_right_arithmetic(xi, 1)
                y = lax.bitcast_convert_type(yi, _F32)
                hx = x * -0.5
                for _ in range(3):
                    y = y * (1.5 + hx * y * y)
                for k in range(KH):
                    sl = pl.ds(k * L, L)
                    xh = (r[k] - meanv) * y
                    rows_v[j, sl] = xh * gamma_v[sl] + beta_v[sl]
                return tcarry

            lax.fori_loop(0, S, tok_body, 0)
            pltpu.sync_copy(rows_v, out_hbm.at[pl.ds(base, S)])
            return carry

        lax.fori_loop(0, chunks, chunk_body, 0)

    return sc_kernel


def kernel(input_ids, token_type_ids, word_table, pos_table, type_table,
           gamma, beta):
    B, S = input_ids.shape
    V, H = word_table.shape
    TOK = B * S
    ids2 = input_ids.reshape(TOK // _IDXW, _IDXW).astype(_I32)
    ttf = token_type_ids.reshape(TOK).astype(_I32)
    fn = _make_sc_kernel(TOK, S, H, V)
    out = fn(ids2, ttf, word_table.astype(_F32), pos_table.astype(_F32),
             type_table.astype(_F32), gamma.astype(_F32), beta.astype(_F32))
    return out.reshape(B, S, H)


# speedup vs baseline: 5.2129x; 1.7518x over previous
import functools

import jax
import jax.numpy as jnp
from jax import lax
from jax.experimental import pallas as pl
from jax.experimental.pallas import tpu as pltpu
from jax.experimental.pallas import tpu_sc as plsc

_F32 = jnp.float32
_I32 = jnp.int32
_EPS = 1e-12
_IDXW = 100


@functools.lru_cache(maxsize=None)
def _make_sc_kernel(TOK, S, H, V):
    info = plsc.get_sparse_core_info()
    NC, NS, L = info.num_cores, info.num_subcores, info.num_lanes
    NW = NC * NS
    assert TOK % (NW * S) == 0
    per_w = TOK // NW
    chunks = per_w // S
    KH = H // L
    assert S % _IDXW == 0
    G = S // _IDXW
    rows_per_w = per_w // _IDXW

    mesh = plsc.VectorSubcoreMesh(core_axis_name="c", subcore_axis_name="s")

    @functools.partial(
        pl.kernel,
        mesh=mesh,
        out_type=jax.ShapeDtypeStruct((TOK, H), _F32),
        scratch_types=[
            pltpu.VMEM((G, _IDXW), _I32),
            pltpu.VMEM((S + 16,), _I32),
            pltpu.VMEM((S, H), _F32),
            pltpu.VMEM((S, H), _F32),
            pltpu.VMEM((2, H), _F32),
            pltpu.VMEM((H,), _F32),
            pltpu.VMEM((H,), _F32),
            pltpu.VMEM((H,), _F32),
            pltpu.SemaphoreType.DMA,
        ],
    )
    def sc_kernel(ids_hbm, tt_hbm, word_hbm, pos_hbm, type_hbm, gamma_hbm,
                  beta_hbm, out_hbm, idx_v, tt_v, posx_v, rows_v, type_v,
                  gamma_v, beta_v, dtv_v, sem):
        wid = lax.axis_index("s") * NC + lax.axis_index("c")

        pltpu.sync_copy(pos_hbm.at[pl.ds(0, S)], posx_v)
        pltpu.sync_copy(type_hbm, type_v)
        pltpu.sync_copy(gamma_hbm, gamma_v)
        pltpu.sync_copy(beta_hbm, beta_v)
        for k in range(KH):
            sl = pl.ds(k * L, L)
            dtv_v[sl] = type_v[1, sl] - type_v[0, sl]

        def fuse_body(j, carry):
            for k in range(KH):
                sl = pl.ds(k * L, L)
                posx_v[j, sl] = posx_v[j, sl] + type_v[0, sl]
            return carry

        lax.fori_loop(0, S, fuse_body, 0)

        def chunk_body(c, carry):
            base = wid * per_w + c * S
            pltpu.sync_copy(tt_hbm.at[pl.ds(base, S)], tt_v.at[pl.ds(0, S)])
            rbase = wid * rows_per_w + c * G
            pltpu.sync_copy(ids_hbm.at[pl.ds(rbase, G)], idx_v)
            copies = [
                pltpu.async_copy(word_hbm.at[idx_v.at[g]],
                                 rows_v.at[pl.ds(g * _IDXW, _IDXW)], sem)
                for g in range(G)
            ]
            for cp in copies:
                cp.wait()

            @plsc.parallel_loop(0, S, 1, unroll=4)
            def tok_body(j):
                ttf = jnp.full((L,), tt_v[pl.ds(j, L)][0].astype(_F32))
                s = jnp.zeros((L,), _F32)
                q = jnp.zeros((L,), _F32)
                r = []
                for k in range(KH):
                    sl = pl.ds(k * L, L)
                    a = rows_v[j, sl] + posx_v[j, sl] + ttf * dtv_v[sl]
                    r.append(a)
                    s = s + a
                    q = q + a * a
                lanes = jnp.arange(L, dtype=_I32)
                for d in (8, 4, 2, 1):
                    perm = lanes ^ d
                    s = s + jnp.take_along_axis(
                        s, perm, axis=0, mode="promise_in_bounds")
                    q = q + jnp.take_along_axis(
                        q, perm, axis=0, mode="promise_in_bounds")
                meanv = s * (1.0 / H)
                qv = q
                varv = qv * (1.0 / H) - meanv * meanv
                x = varv + _EPS
                xi = lax.bitcast_convert_type(x, _I32)
                yi = jnp.int32(0x5F3759DF) - lax.shift_right_arithmetic(xi, 1)
                y = lax.bitcast_convert_type(yi, _F32)
                hx = x * -0.5
                for _ in range(2):
                    y = y * (1.5 + hx * y * y)
                for k in range(KH):
                    sl = pl.ds(k * L, L)
                    xh = (r[k] - meanv) * y
                    rows_v[j, sl] = xh * gamma_v[sl] + beta_v[sl]

            pltpu.sync_copy(rows_v, out_hbm.at[pl.ds(base, S)])
            return carry

        lax.fori_loop(0, chunks, chunk_body, 0)

    return sc_kernel


def kernel(input_ids, token_type_ids, word_table, pos_table, type_table,
           gamma, beta):
    B, S = input_ids.shape
    V, H = word_table.shape
    TOK = B * S
    ids2 = input_ids.reshape(TOK // _IDXW, _IDXW).astype(_I32)
    ttf = token_type_ids.reshape(TOK).astype(_I32)
    fn = _make_sc_kernel(TOK, S, H, V)
    out = fn(ids2, ttf, word_table.astype(_F32), pos_table.astype(_F32),
             type_table.astype(_F32), gamma.astype(_F32), beta.astype(_F32))
    return out.reshape(B, S, H)


# speedup vs baseline: 7.4473x; 1.4286x over previous
import functools

import jax
import jax.numpy as jnp
from jax import lax
from jax.experimental import pallas as pl
from jax.experimental.pallas import tpu as pltpu
from jax.experimental.pallas import tpu_sc as plsc

_F32 = jnp.float32
_I32 = jnp.int32
_EPS = 1e-12
_IDXW = 100


@functools.lru_cache(maxsize=None)
def _make_sc_kernel(TOK, S, H, V):
    info = plsc.get_sparse_core_info()
    NC, NS, L = info.num_cores, info.num_subcores, info.num_lanes
    NW = NC * NS
    assert TOK % (NW * S) == 0
    per_w = TOK // NW
    chunks = per_w // S
    KH = H // L
    assert S % _IDXW == 0
    G = S // _IDXW
    rows_per_w = per_w // _IDXW

    mesh = plsc.VectorSubcoreMesh(core_axis_name="c", subcore_axis_name="s")

    @functools.partial(
        pl.kernel,
        mesh=mesh,
        out_type=jax.ShapeDtypeStruct((TOK, H), _F32),
        scratch_types=[
            pltpu.VMEM((G, _IDXW), _I32),
            pltpu.VMEM((S + 16,), _I32),
            pltpu.VMEM((S, H), _F32),
            pltpu.VMEM((S, H), _F32),
            pltpu.VMEM((2, H), _F32),
            pltpu.VMEM((H,), _F32),
            pltpu.VMEM((H,), _F32),
            pltpu.VMEM((H,), _F32),
            pltpu.SemaphoreType.DMA,
        ],
    )
    def sc_kernel(ids_hbm, tt_hbm, word_hbm, pos_hbm, type_hbm, gamma_hbm,
                  beta_hbm, out_hbm, idx_v, tt_v, posx_v, rows_v, type_v,
                  gamma_v, beta_v, dtv_v, sem):
        wid = lax.axis_index("s") * NC + lax.axis_index("c")

        pltpu.sync_copy(pos_hbm.at[pl.ds(0, S)], posx_v)
        pltpu.sync_copy(type_hbm, type_v)
        pltpu.sync_copy(gamma_hbm, gamma_v)
        pltpu.sync_copy(beta_hbm, beta_v)
        for k in range(KH):
            sl = pl.ds(k * L, L)
            dtv_v[sl] = type_v[1, sl] - type_v[0, sl]

        def fuse_body(j, carry):
            for k in range(KH):
                sl = pl.ds(k * L, L)
                posx_v[j, sl] = posx_v[j, sl] + type_v[0, sl]
            return carry

        lax.fori_loop(0, S, fuse_body, 0)

        def chunk_body(c, carry):
            base = wid * per_w + c * S
            pltpu.sync_copy(tt_hbm.at[pl.ds(base, S)], tt_v.at[pl.ds(0, S)])
            rbase = wid * rows_per_w + c * G
            pltpu.sync_copy(ids_hbm.at[pl.ds(rbase, G)], idx_v)
            copies = [
                pltpu.async_copy(word_hbm.at[idx_v.at[g]],
                                 rows_v.at[pl.ds(g * _IDXW, _IDXW)], sem)
                for g in range(G)
            ]
            for cp in copies:
                cp.wait()

            @plsc.parallel_loop(0, S, 1, unroll=1)
            def tok_body(j):
                ttf = jnp.full((L,), tt_v[pl.ds(j, L)][0].astype(_F32))
                s = jnp.zeros((L,), _F32)
                q = jnp.zeros((L,), _F32)
                r = []
                for k in range(KH):
                    sl = pl.ds(k * L, L)
                    a = rows_v[j, sl] + posx_v[j, sl] + ttf * dtv_v[sl]
                    r.append(a)
                    s = s + a
                    q = q + a * a
                lanes = jnp.arange(L, dtype=_I32)
                for d in (8, 4, 2, 1):
                    perm = lanes ^ d
                    s = s + jnp.take_along_axis(
                        s, perm, axis=0, mode="promise_in_bounds")
                    q = q + jnp.take_along_axis(
                        q, perm, axis=0, mode="promise_in_bounds")
                meanv = s * (1.0 / H)
                qv = q
                varv = qv * (1.0 / H) - meanv * meanv
                x = varv + _EPS
                xi = lax.bitcast_convert_type(x, _I32)
                yi = jnp.int32(0x5F3759DF) - lax.shift_right_arithmetic(xi, 1)
                y = lax.bitcast_convert_type(yi, _F32)
                hx = x * -0.5
                for _ in range(2):
                    y = y * (1.5 + hx * y * y)
                for k in range(KH):
                    sl = pl.ds(k * L, L)
                    xh = (r[k] - meanv) * y
                    rows_v[j, sl] = xh * gamma_v[sl] + beta_v[sl]

            pltpu.sync_copy(rows_v, out_hbm.at[pl.ds(base, S)])
            return carry

        lax.fori_loop(0, chunks, chunk_body, 0)

    return sc_kernel


def kernel(input_ids, token_type_ids, word_table, pos_table, type_table,
           gamma, beta):
    B, S = input_ids.shape
    V, H = word_table.shape
    TOK = B * S
    ids2 = input_ids.reshape(TOK // _IDXW, _IDXW).astype(_I32)
    ttf = token_type_ids.reshape(TOK).astype(_I32)
    fn = _make_sc_kernel(TOK, S, H, V)
    out = fn(ids2, ttf, word_table.astype(_F32), pos_table.astype(_F32),
             type_table.astype(_F32), gamma.astype(_F32), beta.astype(_F32))
    return out.reshape(B, S, H)
